# baseline (speedup 1.0000x reference)
import jax
import jax.numpy as jnp
from jax import lax
from jax.experimental import pallas as pl
from jax.experimental.pallas import tpu as pltpu
from jax.experimental.pallas import tpu_sc as plsc

N = 10000
E = 320000
D = 128
ALPHA = 0.2

NC = 2
NS = 16
NW = NC * NS
EPW = E // NW
C = 80
CH = EPW // C
NP = 10240
RP = NP // NS
NR = 2
NI = 3

_f32 = jnp.float32


def _stage_a(h, W, a_mat):
    BN = 400

    def body(h_ref, w_ref, am_ref, wh_ref, s_ref):
        wh = jnp.dot(h_ref[...], w_ref[...], preferred_element_type=_f32)
        wh_ref[...] = wh
        s_ref[...] = jnp.dot(wh, am_ref[...], preferred_element_type=_f32)

    return pl.pallas_call(
        body,
        grid=(N // BN,),
        in_specs=[
            pl.BlockSpec((BN, D), lambda i: (i, 0)),
            pl.BlockSpec((D, D), lambda i: (0, 0)),
            pl.BlockSpec((D, 2), lambda i: (0, 0)),
        ],
        out_specs=[
            pl.BlockSpec((BN, D), lambda i: (i, 0)),
            pl.BlockSpec((BN, 2), lambda i: (i, 0)),
        ],
        out_shape=[
            jax.ShapeDtypeStruct((N, D), _f32),
            jax.ShapeDtypeStruct((N, 2), _f32),
        ],
    )(h, W, a_mat)


def _sc_body(wh_hbm, s12_hbm, adj_hbm, hs_out, den_out,
             s12_v, pairs, dsti, srci, xe_v, rows, tmp_v, zrow_v,
             hs_sh, den_sh, sem_gat, sem_idx, sem_sca, sem_den):
    cid = lax.axis_index("c")
    sid = lax.axis_index("s")
    wid = cid * NS + sid

    zeros16 = jnp.zeros((16,), _f32)

    def ztmp(i, carry):
        for k in range(D // 16):
            tmp_v[i, pl.ds(k * 16, 16)] = zeros16
        return carry

    lax.fori_loop(0, 40, ztmp, 0)

    def zrow(j, carry):
        zrow_v[pl.ds(j * 16, 16)] = zeros16
        return carry

    lax.fori_loop(0, RP // 16, zrow, 0)

    base_r = sid * RP
    for kc in range(RP // 40):
        pltpu.sync_copy(tmp_v, hs_sh.at[pl.ds(base_r + kc * 40, 40)])
    pltpu.sync_copy(zrow_v, den_sh.at[pl.ds(base_r, RP)])
    pltpu.sync_copy(s12_hbm, s12_v)
    plsc.subcore_barrier()

    ebase = wid * EPW
    iot2 = jnp.arange(16, dtype=jnp.int32) * 2

    def issue_idx(q, ji):
        eb = 2 * (ebase + q * C)
        pltpu.async_copy(adj_hbm.at[pl.ds(eb, 2 * C)],
                         pairs.at[pl.ds(ji * 2 * C, 2 * C)],
                         sem_idx.at[ji])

    def wait_idx(q, ji):
        eb = 2 * (ebase + q * C)
        pltpu.make_async_copy(adj_hbm.at[pl.ds(eb, 2 * C)],
                              pairs.at[pl.ds(ji * 2 * C, 2 * C)],
                              sem_idx.at[ji]).wait()

    def deinterleave(ji):
        for g in range(C // 16):
            sl = pl.ds(g * 16, 16)
            base = ji * 2 * C + 32 * g
            dv = plsc.load_gather(pairs, [iot2 + base])
            sv = plsc.load_gather(pairs, [iot2 + base + 1])
            dsti[ji, sl] = dv
            srci[ji, sl] = sv

    def issue_gather(jr, ji):
        pltpu.async_copy(wh_hbm.at[srci.at[ji]], rows.at[jr],
                         sem_gat.at[jr])

    def wait_gather(jr, ji):
        pltpu.make_async_copy(wh_hbm.at[srci.at[ji]], rows.at[jr],
                              sem_gat.at[jr]).wait()

    def drain_scatters(jr, ji):
        pltpu.make_async_copy(rows.at[jr], hs_sh.at[dsti.at[ji]],
                              sem_sca).wait()
        pltpu.make_async_copy(xe_v, den_sh.at[dsti.at[ji]],
                              sem_den).wait()

    pltpu.sync_copy(adj_hbm.at[pl.ds(2 * ebase, 2 * C)],
                    pairs.at[pl.ds(0, 2 * C)])
    deinterleave(0)
    issue_gather(0, 0)
    issue_idx(1, 1)

    def step(t, carry):
        for u in range(6):
            q = t * 6 + u
            jr = u % NR
            ji = u % NI
            jr1 = (u + 1) % NR
            ji1 = (u + 1) % NI
            ji2 = (u + 2) % NI

            @pl.when(q <= CH - 1)
            def _w2():
                wait_gather(jr, ji)

            @pl.when(jnp.logical_and(q >= 1, q <= CH - 1))
            def _w1():
                drain_scatters(jr1, (u - 1) % NI)

            @pl.when(q + 1 <= CH - 1)
            def _w3():
                wait_idx(q + 1, ji1)
                deinterleave(ji1)
                issue_gather(jr1, ji1)

            @pl.when(q + 2 <= CH - 1)
            def _w4():
                issue_idx(q + 2, ji2)

            @pl.when(q <= CH - 1)
            def _w5():
                for g in range(C // 16):
                    sl = pl.ds(g * 16, 16)
                    dv = dsti[ji, sl]
                    sv = srci[ji, sl]
                    s1 = plsc.load_gather(s12_v, [dv * 2])
                    s2 = plsc.load_gather(s12_v, [sv * 2 + 1])
                    x = s1 + s2
                    x = jnp.where(x > 0, x, x * ALPHA)
                    xe_v[sl] = jnp.minimum(jnp.exp(x), 1e6)

                def scale(g, carry2):
                    xev = xe_v[pl.ds(g * 16, 16)]
                    for lane in range(16):
                        xs = xev[lane]
                        e = g * 16 + lane
                        for k in range(D // 16):
                            sl2 = pl.ds(k * 16, 16)
                            rows[jr, e, sl2] = rows[jr, e, sl2] * xs
                    return carry2

                lax.fori_loop(0, C // 16, scale, 0)
                pltpu.async_copy(rows.at[jr], hs_sh.at[dsti.at[ji]],
                                 sem_sca, add=True)
                pltpu.async_copy(xe_v, den_sh.at[dsti.at[ji]],
                                 sem_den, add=True)

        return carry

    lax.fori_loop(0, (CH + 5) // 6, step, 0)
    drain_scatters((CH - 1) % NR, (CH - 1) % NI)
    plsc.subcore_barrier()

    out_r = cid * NP + base_r
    pltpu.sync_copy(hs_sh.at[pl.ds(base_r, RP)],
                    hs_out.at[pl.ds(out_r, RP)])
    pltpu.sync_copy(den_sh.at[pl.ds(base_r, RP)],
                    den_out.at[pl.ds(out_r, RP)])


def _stage_b(Wh, s12, adj_flat):
    mesh = plsc.VectorSubcoreMesh(
        core_axis_name="c", subcore_axis_name="s",
        num_cores=NC, num_subcores=NS)
    i32 = jnp.int32
    kb = pl.kernel(
        _sc_body,
        out_type=[
            jax.ShapeDtypeStruct((NC * NP, D), _f32),
            jax.ShapeDtypeStruct((NC * NP,), _f32),
        ],
        mesh=mesh,
        compiler_params=pltpu.CompilerParams(needs_layout_passes=False),
        scratch_types=[
            pltpu.VMEM((2 * N,), _f32),
            pltpu.VMEM((NI * 2 * C,), i32),
            pltpu.VMEM((NI, C), i32),
            pltpu.VMEM((NI, C), i32),
            pltpu.VMEM((C,), _f32),
            pltpu.VMEM((NR, C, D), _f32),
            pltpu.VMEM((40, D), _f32),
            pltpu.VMEM((RP,), _f32),
            pltpu.VMEM_SHARED((NP, D), _f32),
            pltpu.VMEM_SHARED((NP,), _f32),
            pltpu.SemaphoreType.DMA((NR,)),
            pltpu.SemaphoreType.DMA((NI,)),
            pltpu.SemaphoreType.DMA,
            pltpu.SemaphoreType.DMA,
        ],
    )
    return kb(Wh, s12, adj_flat)


def _stage_c(hs, den):
    BN = 400

    def body(hs_ref, den_ref, out_ref):
        p = hs_ref[0] + hs_ref[1]
        d = den_ref[0] + den_ref[1] + 1e-10
        v = p * (1.0 / d)
        out_ref[...] = jnp.where(v > 0, v, jnp.exp(v) - 1.0)

    return pl.pallas_call(
        body,
        grid=(N // BN,),
        in_specs=[
            pl.BlockSpec((2, BN, D), lambda i: (0, i, 0)),
            pl.BlockSpec((2, BN, 1), lambda i: (0, i, 0)),
        ],
        out_specs=pl.BlockSpec((BN, D), lambda i: (i, 0)),
        out_shape=jax.ShapeDtypeStruct((N, D), _f32),
    )(hs, den)


def kernel(h, adj, W, a):
    adj_flat = adj.astype(jnp.int32).reshape(2 * E)
    a_mat = jnp.concatenate([a[:D], a[D:]], axis=1)
    Wh, s12 = _stage_a(h, W, a_mat)
    s12_flat = s12.reshape(2 * N)
    hs, den = _stage_b(Wh, s12_flat, adj_flat)
    return _stage_c(hs.reshape(NC, NP, D), den.reshape(NC, NP, 1))


# speedup vs baseline: 1.6873x; 1.6873x over previous
import jax
import jax.numpy as jnp
from jax import lax
from jax.experimental import pallas as pl
from jax.experimental.pallas import tpu as pltpu
from jax.experimental.pallas import tpu_sc as plsc

N = 10000
E = 320000
D = 128
ALPHA = 0.2

NC = 2
NS = 16
NW = NC * NS
EPW = E // NW
C = 80
CH = EPW // C
NP = 10240
RP = NP // NS
NR = 2
NI = 3

_f32 = jnp.float32


def _stage_a(h, W, a_mat):
    BN = 400

    def body(h_ref, w_ref, am_ref, wh_ref, s_ref):
        wh = jnp.dot(h_ref[...], w_ref[...], preferred_element_type=_f32)
        wh_ref[...] = wh
        s_ref[...] = jnp.dot(wh, am_ref[...], preferred_element_type=_f32)

    return pl.pallas_call(
        body,
        grid=(N // BN,),
        in_specs=[
            pl.BlockSpec((BN, D), lambda i: (i, 0)),
            pl.BlockSpec((D, D), lambda i: (0, 0)),
            pl.BlockSpec((D, 2), lambda i: (0, 0)),
        ],
        out_specs=[
            pl.BlockSpec((BN, D), lambda i: (i, 0)),
            pl.BlockSpec((BN, 2), lambda i: (i, 0)),
        ],
        out_shape=[
            jax.ShapeDtypeStruct((N, D), _f32),
            jax.ShapeDtypeStruct((N, 2), _f32),
        ],
    )(h, W, a_mat)


def _sc_body(wh_hbm, s12_hbm, dst_hbm, src_hbm, hs_out, den_out,
             s12_v, dsti, srci, xe_v, rows, tmp_v, zrow_v,
             hs_sh, den_sh, sem_gat, sem_idx, sem_sca, sem_den):
    cid = lax.axis_index("c")
    sid = lax.axis_index("s")
    wid = cid * NS + sid

    zeros16 = jnp.zeros((16,), _f32)

    def ztmp(i, carry):
        for k in range(D // 16):
            tmp_v[i, pl.ds(k * 16, 16)] = zeros16
        return carry

    lax.fori_loop(0, 40, ztmp, 0)

    def zrow(j, carry):
        zrow_v[pl.ds(j * 16, 16)] = zeros16
        return carry

    lax.fori_loop(0, RP // 16, zrow, 0)

    base_r = sid * RP
    for kc in range(RP // 40):
        pltpu.sync_copy(tmp_v, hs_sh.at[pl.ds(base_r + kc * 40, 40)])
    pltpu.sync_copy(zrow_v, den_sh.at[pl.ds(base_r, RP)])
    pltpu.sync_copy(s12_hbm, s12_v)
    plsc.subcore_barrier()

    ebase = wid * EPW

    def issue_idx(q, ji):
        eb = ebase + q * C
        pltpu.async_copy(dst_hbm.at[pl.ds(eb, C)], dsti.at[ji],
                         sem_idx.at[ji])
        pltpu.async_copy(src_hbm.at[pl.ds(eb, C)], srci.at[ji],
                         sem_idx.at[ji])

    def wait_idx(q, ji):
        eb = ebase + q * C
        pltpu.make_async_copy(dst_hbm.at[pl.ds(eb, C)], dsti.at[ji],
                              sem_idx.at[ji]).wait()
        pltpu.make_async_copy(src_hbm.at[pl.ds(eb, C)], srci.at[ji],
                              sem_idx.at[ji]).wait()

    def issue_gather(jr, ji):
        pltpu.async_copy(wh_hbm.at[srci.at[ji]], rows.at[jr],
                         sem_gat.at[jr])

    def wait_gather(jr, ji):
        pltpu.make_async_copy(wh_hbm.at[srci.at[ji]], rows.at[jr],
                              sem_gat.at[jr]).wait()

    def drain_scatters(jr, ji):
        pltpu.make_async_copy(rows.at[jr], hs_sh.at[dsti.at[ji]],
                              sem_sca).wait()
        pltpu.make_async_copy(xe_v, den_sh.at[dsti.at[ji]],
                              sem_den).wait()

    pltpu.sync_copy(dst_hbm.at[pl.ds(ebase, C)], dsti.at[0])
    pltpu.sync_copy(src_hbm.at[pl.ds(ebase, C)], srci.at[0])
    issue_gather(0, 0)
    issue_idx(1, 1)

    def step(t, carry):
        for u in range(6):
            q = t * 6 + u
            jr = u % NR
            ji = u % NI
            jr1 = (u + 1) % NR
            ji1 = (u + 1) % NI
            ji2 = (u + 2) % NI

            @pl.when(q <= CH - 1)
            def _w2():
                wait_gather(jr, ji)

            @pl.when(jnp.logical_and(q >= 1, q <= CH - 1))
            def _w1():
                drain_scatters(jr1, (u - 1) % NI)

            @pl.when(q + 1 <= CH - 1)
            def _w3():
                wait_idx(q + 1, ji1)
                issue_gather(jr1, ji1)

            @pl.when(q + 2 <= CH - 1)
            def _w4():
                issue_idx(q + 2, ji2)

            @pl.when(q <= CH - 1)
            def _w5():
                for g in range(C // 16):
                    sl = pl.ds(g * 16, 16)
                    dv = dsti[ji, sl]
                    sv = srci[ji, sl]
                    s1 = plsc.load_gather(s12_v, [dv * 2])
                    s2 = plsc.load_gather(s12_v, [sv * 2 + 1])
                    x = s1 + s2
                    x = jnp.where(x > 0, x, x * ALPHA)
                    xe_v[sl] = jnp.minimum(jnp.exp(x), 1e6)

                def scale(g, carry2):
                    xev = xe_v[pl.ds(g * 16, 16)]
                    for lane in range(16):
                        xs = xev[lane]
                        e = g * 16 + lane
                        for k in range(D // 16):
                            sl2 = pl.ds(k * 16, 16)
                            rows[jr, e, sl2] = rows[jr, e, sl2] * xs
                    return carry2

                lax.fori_loop(0, C // 16, scale, 0)
                pltpu.async_copy(rows.at[jr], hs_sh.at[dsti.at[ji]],
                                 sem_sca, add=True)
                pltpu.async_copy(xe_v, den_sh.at[dsti.at[ji]],
                                 sem_den, add=True)

        return carry

    lax.fori_loop(0, (CH + 5) // 6, step, 0)
    drain_scatters((CH - 1) % NR, (CH - 1) % NI)
    plsc.subcore_barrier()

    out_r = cid * NP + base_r
    pltpu.sync_copy(hs_sh.at[pl.ds(base_r, RP)],
                    hs_out.at[pl.ds(out_r, RP)])
    pltpu.sync_copy(den_sh.at[pl.ds(base_r, RP)],
                    den_out.at[pl.ds(out_r, RP)])


def _stage_b(Wh, s12, dst, src):
    mesh = plsc.VectorSubcoreMesh(
        core_axis_name="c", subcore_axis_name="s",
        num_cores=NC, num_subcores=NS)
    i32 = jnp.int32
    kb = pl.kernel(
        _sc_body,
        out_type=[
            jax.ShapeDtypeStruct((NC * NP, D), _f32),
            jax.ShapeDtypeStruct((NC * NP,), _f32),
        ],
        mesh=mesh,
        compiler_params=pltpu.CompilerParams(needs_layout_passes=False),
        scratch_types=[
            pltpu.VMEM((2 * N,), _f32),
            pltpu.VMEM((NI, C), i32),
            pltpu.VMEM((NI, C), i32),
            pltpu.VMEM((C,), _f32),
            pltpu.VMEM((NR, C, D), _f32),
            pltpu.VMEM((40, D), _f32),
            pltpu.VMEM((RP,), _f32),
            pltpu.VMEM_SHARED((NP, D), _f32),
            pltpu.VMEM_SHARED((NP,), _f32),
            pltpu.SemaphoreType.DMA((NR,)),
            pltpu.SemaphoreType.DMA((NI,)),
            pltpu.SemaphoreType.DMA,
            pltpu.SemaphoreType.DMA,
        ],
    )
    return kb(Wh, s12, dst, src)


def _stage_c(hs, den):
    BN = 400

    def body(hs_ref, den_ref, out_ref):
        p = hs_ref[0] + hs_ref[1]
        d = den_ref[0] + den_ref[1] + 1e-10
        v = p * (1.0 / d)
        out_ref[...] = jnp.where(v > 0, v, jnp.exp(v) - 1.0)

    return pl.pallas_call(
        body,
        grid=(N // BN,),
        in_specs=[
            pl.BlockSpec((2, BN, D), lambda i: (0, i, 0)),
            pl.BlockSpec((2, BN, 1), lambda i: (0, i, 0)),
        ],
        out_specs=pl.BlockSpec((BN, D), lambda i: (i, 0)),
        out_shape=jax.ShapeDtypeStruct((N, D), _f32),
    )(hs, den)


def kernel(h, adj, W, a):
    adj32 = adj.astype(jnp.int32)
    dst = adj32[:, 0]
    src = adj32[:, 1]
    a_mat = jnp.concatenate([a[:D], a[D:]], axis=1)
    Wh, s12 = _stage_a(h, W, a_mat)
    s12_flat = s12.reshape(2 * N)
    hs, den = _stage_b(Wh, s12_flat, dst, src)
    return _stage_c(hs.reshape(NC, NP, D), den.reshape(NC, NP, 1))
